# baseline (speedup 1.0000x reference)
import functools
import math

import jax
import jax.numpy as jnp
from jax import lax
from jax.experimental import pallas as pl
from jax.experimental.pallas import tpu as pltpu
from jax.experimental.pallas import tpu_sc as plsc

N = 4096
E = 131072
H = 256
G = 16
L = 4
ROW_BLK = 512
N_BLKS = N // ROW_BLK
NW = 32
EPT = E // NW
_SC_MESH = plsc.VectorSubcoreMesh(core_axis_name="c", subcore_axis_name="s")
_SC_PARAMS = pltpu.CompilerParams(needs_layout_passes=False)


def _ln_in(x, g, b):
    m = jnp.mean(x, axis=-1, keepdims=True)
    v = jnp.mean((x - m) ** 2, axis=-1, keepdims=True)
    return (x - m) * jax.lax.rsqrt(v + 1e-5) * g + b


def _dot3(a, b, dims):
    ah = a.astype(jnp.bfloat16)
    al = (a - ah.astype(jnp.float32)).astype(jnp.bfloat16)
    bh = b.astype(jnp.bfloat16)
    bl = (b - bh.astype(jnp.float32)).astype(jnp.bfloat16)

    def d(x, y):
        return jax.lax.dot_general(
            x, y, dims, precision=jax.lax.Precision.DEFAULT,
            preferred_element_type=jnp.float32)

    return d(ah, bh) + (d(ah, bl) + d(al, bh))


def _mm(a, b):
    return _dot3(a, b, (((1,), (0,)), ((), ())))


def _proj_body(x_ref, w_ref, b_ref, o_ref):
    o_ref[...] = _mm(x_ref[...], w_ref[...]) + b_ref[...]


def _input_proj(x, w, b):
    return pl.pallas_call(
        _proj_body,
        grid=(N_BLKS,),
        in_specs=[
            pl.BlockSpec((ROW_BLK, H), lambda i: (i, 0)),
            pl.BlockSpec((H, H), lambda i: (0, 0)),
            pl.BlockSpec((1, H), lambda i: (0, 0)),
        ],
        out_specs=pl.BlockSpec((ROW_BLK, H), lambda i: (i, 0)),
        out_shape=jax.ShapeDtypeStruct((N, H), jnp.float32),
    )(x, w, b.reshape(1, H))


def _qkv_body(h_ref, g_ref, b_ref, wq_ref, wk_ref, wv_ref,
              q_ref, k_ref, v_ref):
    xn = _ln_in(h_ref[...], g_ref[...], b_ref[...])
    q_ref[...] = _mm(xn, wq_ref[...])
    k_ref[...] = _mm(xn, wk_ref[...])
    v_ref[...] = _mm(xn, wv_ref[...])


def _qkv(h, g, b, wq, wk, wv):
    blk = pl.BlockSpec((ROW_BLK, H), lambda i: (i, 0))
    wblk = pl.BlockSpec((H, H), lambda i: (0, 0))
    vec = pl.BlockSpec((1, H), lambda i: (0, 0))
    return pl.pallas_call(
        _qkv_body,
        grid=(N_BLKS,),
        in_specs=[blk, vec, vec, wblk, wblk, wblk],
        out_specs=[blk, blk, blk],
        out_shape=[jax.ShapeDtypeStruct((N, H), jnp.float32)] * 3,
    )(h, g.reshape(1, H), b.reshape(1, H), wq, wk, wv)


def _score_body(q_ref, k_ref, s_ref):
    s_ref[...] = _dot3(q_ref[...], k_ref[...],
                       (((1,), (1,)), ((), ()))).astype(jnp.bfloat16)


def _scores(q, k):
    return pl.pallas_call(
        _score_body,
        grid=(N_BLKS,),
        in_specs=[
            pl.BlockSpec((ROW_BLK, H), lambda i: (i, 0)),
            pl.BlockSpec((N, H), lambda i: (0, 0)),
        ],
        out_specs=pl.BlockSpec((ROW_BLK, N), lambda i: (i, 0)),
        out_shape=jax.ShapeDtypeStruct((N, N), jnp.bfloat16),
    )(q, k)


def _ffn_body(h_ref, a0_ref, a1_ref, d0_ref, d1_ref, g_ref, b_ref,
              w1_ref, b1_ref, w2_ref, b2_ref, o_ref):
    den = d0_ref[0] + d1_ref[0] + 1e-16
    t = h_ref[...] + (a0_ref[0] + a1_ref[0]) / den.reshape(ROW_BLK, 1)
    xn = _ln_in(t, g_ref[...], b_ref[...])
    u = _mm(xn, w1_ref[...]) + b1_ref[...]
    act = 0.5 * u * (1.0 + jax.lax.erf(u * (2.0 ** -0.5)))
    o_ref[...] = t + _mm(act, w2_ref[...]) + b2_ref[...]


def _ffn(h, agg_parts, den_parts, g, b, w1, b1, w2, b2):
    blk = pl.BlockSpec((ROW_BLK, H), lambda i: (i, 0))
    vec = pl.BlockSpec((1, H), lambda i: (0, 0))
    return pl.pallas_call(
        _ffn_body,
        grid=(N_BLKS,),
        in_specs=[
            blk,
            pl.BlockSpec((1, ROW_BLK, H), lambda i: (0, i, 0)),
            pl.BlockSpec((1, ROW_BLK, H), lambda i: (0, i, 0)),
            pl.BlockSpec((1, ROW_BLK), lambda i: (0, i)),
            pl.BlockSpec((1, ROW_BLK), lambda i: (0, i)),
            vec, vec,
            pl.BlockSpec((H, 2 * H), lambda i: (0, 0)),
            pl.BlockSpec((1, 2 * H), lambda i: (0, 0)),
            pl.BlockSpec((2 * H, H), lambda i: (0, 0)),
            vec,
        ],
        out_specs=blk,
        out_shape=jax.ShapeDtypeStruct((N, H), jnp.float32),
    )(h, agg_parts[0][None], agg_parts[1][None],
      den_parts[0][None], den_parts[1][None],
      g.reshape(1, H), b.reshape(1, H),
      w1, b1.reshape(1, 2 * H), w2, b2.reshape(1, H))


def _head_body(h_ref, batch_ref, wc1_ref, bc1_ref, wc2_ref, bc2_ref, o_ref):
    onehot = (batch_ref[...] == jax.lax.broadcasted_iota(
        jnp.int32, (N, G), 1)).astype(jnp.float32)
    sums = jax.lax.dot_general(
        onehot, h_ref[...], (((0,), (0,)), ((), ())),
        precision=jax.lax.Precision.HIGHEST,
        preferred_element_type=jnp.float32)
    cnt = jnp.sum(onehot, axis=0, keepdims=True)
    gmean = sums / jnp.clip(cnt, 1.0, None).reshape(G, 1)
    z = jnp.maximum(_mm(gmean, wc1_ref[...]) + bc1_ref[...], 0.0)
    o_ref[...] = _mm(z, wc2_ref[...]) + bc2_ref[...]


def _head(h, batch, wc1, bc1, wc2, bc2):
    OUT = wc2.shape[1]
    return pl.pallas_call(
        _head_body,
        in_specs=[
            pl.BlockSpec((N, H), lambda: (0, 0)),
            pl.BlockSpec((N, 1), lambda: (0, 0)),
            pl.BlockSpec(wc1.shape, lambda: (0, 0)),
            pl.BlockSpec((1, wc1.shape[1]), lambda: (0, 0)),
            pl.BlockSpec(wc2.shape, lambda: (0, 0)),
            pl.BlockSpec((1, OUT), lambda: (0, 0)),
        ],
        out_specs=pl.BlockSpec((G, OUT), lambda: (0, 0)),
        out_shape=jax.ShapeDtypeStruct((G, OUT), jnp.float32),
    )(h, batch.reshape(N, 1), wc1, bc1.reshape(1, -1), wc2,
      bc2.reshape(1, -1))


import functools


@functools.partial(
    pl.kernel,
    out_type=[jax.ShapeDtypeStruct((NW, EPT), jnp.int32)] +
             [jax.ShapeDtypeStruct((NW, EPT), jnp.float32)] * L,
    mesh=_SC_MESH,
    scratch_types=[
        pltpu.VMEM((EPT,), jnp.int32),
        pltpu.VMEM((EPT,), jnp.int32),
        pltpu.VMEM((128,), jnp.int32),
        pltpu.VMEM((128,), jnp.int32),
        pltpu.VMEM((128,), jnp.int32),
        pltpu.VMEM((EPT,), jnp.int32),
        pltpu.VMEM((L * 16,), jnp.float32),
        pltpu.VMEM((L * 64,), jnp.float32),
        pltpu.VMEM((EPT,), jnp.float32),
        pltpu.VMEM((EPT,), jnp.float32),
        pltpu.VMEM((EPT,), jnp.float32),
        pltpu.VMEM((EPT,), jnp.float32),
        pltpu.SemaphoreType.DMA,
    ],
    compiler_params=_SC_PARAMS,
)
def _sc_prep(p_hbm, d_hbm, row_hbm, col_hbm, fp_hbm, fd_hbm,
             sidx_out, b0_out, b1_out, b2_out, b3_out,
             row_v, col_v, idx_v, p_v, d_v, sidx_v, fp_v, fd_v,
             b0_v, b1_v, b2_v, b3_v, sem):
    c = lax.axis_index("c")
    s = lax.axis_index("s")
    wid = c * 16 + s
    b_refs = (b0_v, b1_v, b2_v, b3_v)
    b_outs = (b0_out, b1_out, b2_out, b3_out)
    pltpu.sync_copy(row_hbm.at[wid], row_v)
    pltpu.sync_copy(col_hbm.at[wid], col_v)
    pltpu.sync_copy(fp_hbm, fp_v)
    pltpu.sync_copy(fd_hbm, fd_v)

    def chunk(ci, carry):
        base = ci * 128
        for g in range(8):
            off = base + g * 16
            r16 = row_v[pl.ds(off, 16)]
            c16 = col_v[pl.ds(off, 16)]
            idx_v[pl.ds(g * 16, 16)] = r16 * N + c16
            sidx_v[pl.ds(off, 16)] = lax.shift_right_logical(
                c16 * N + r16, 1)
        pltpu.async_copy(p_hbm.at[idx_v], p_v, sem).wait()
        pltpu.async_copy(d_hbm.at[idx_v], d_v, sem).wait()
        for g in range(8):
            off = base + g * 16
            p16 = p_v[pl.ds(g * 16, 16)]
            d16 = d_v[pl.ds(g * 16, 16)]
            for l in range(L):
                bv = (plsc.load_gather(fp_v, [p16 + (l * 16)]) +
                      plsc.load_gather(fd_v, [d16 + (l * 64)]))
                b_refs[l][pl.ds(off, 16)] = bv
        return carry

    lax.fori_loop(0, EPT // 128, chunk, 0)
    pltpu.sync_copy(sidx_v, sidx_out.at[wid])
    for l in range(L):
        pltpu.sync_copy(b_refs[l], b_outs[l].at[wid])


_VCH = 64


@functools.partial(
    pl.kernel,
    out_type=[
        jax.ShapeDtypeStruct((2, N, 2, 128), jnp.float32),
        jax.ShapeDtypeStruct((2, N), jnp.float32),
    ],
    mesh=_SC_MESH,
    scratch_types=[
        pltpu.VMEM((EPT,), jnp.int32),
        pltpu.VMEM((EPT,), jnp.int32),
        pltpu.VMEM((EPT,), jnp.int32),
        pltpu.VMEM((16,), jnp.int32),
        pltpu.VMEM((EPT,), jnp.float32),
        pltpu.VMEM((EPT,), jnp.float32),
        pltpu.VMEM((EPT,), jnp.int32),
        pltpu.VMEM((N,), jnp.float32),
        pltpu.VMEM((16, 2, 128), jnp.float32),
        pltpu.VMEM((16, 2, 128), jnp.float32),
        pltpu.VMEM((256,), jnp.float32),
        pltpu.VMEM((256,), jnp.float32),
        pltpu.VMEM_SHARED((N, 2, 128), jnp.float32),
        pltpu.VMEM_SHARED((16, N), jnp.float32),
        pltpu.SemaphoreType.DMA,
        pltpu.SemaphoreType.DMA,
    ],
    compiler_params=_SC_PARAMS,
)
def _sc_edge(s_hbm, v_hbm, row_hbm, col_hbm, sidx_hbm, bias_hbm,
             agg_out, den_out,
             sidx_v, row_v, col_v, col64_v, bias_v, e_v, s_v, den_v,
             vb_a, vb_b, tmp_v, acc_v, agg_sp, den_sp, sem_a, sem_b):
    c = lax.axis_index("c")
    s = lax.axis_index("s")
    wid = c * 16 + s
    pltpu.sync_copy(sidx_hbm.at[wid], sidx_v)
    pltpu.sync_copy(row_hbm.at[wid], row_v)
    pltpu.sync_copy(col_hbm.at[wid], col_v)
    pltpu.sync_copy(bias_hbm.at[wid], bias_v)

    def zden(i, carry):
        den_v[pl.ds(i * 16, 16)] = jnp.zeros((16,), jnp.float32)
        return carry
    lax.fori_loop(0, N // 16, zden, 0)

    for i in range(16):
        for hh in range(2):
            for k in range(8):
                vb_a[i, hh, pl.ds(k * 16, 16)] = jnp.zeros(
                    (16,), jnp.float32)
    for k in range(256 // 16):
        pltpu.sync_copy(vb_a, agg_sp.at[pl.ds(s * 256 + k * 16, 16)])
    plsc.subcore_barrier()

    NCA = EPT // 128

    def _fire_a(ci, sm):
        pltpu.async_copy(s_hbm.at[sidx_v.at[pl.ds(ci * 128, 128)]],
                         s_v.at[pl.ds(ci * 128, 128)], sm)

    def _proc_a(base):
        for g in range(8):
            off = base + g * 16
            pair16 = s_v[pl.ds(off, 16)]
            lo16, hi16 = plsc.unpack(
                plsc.bitcast(pair16, jnp.bfloat16),
                format=plsc.PackFormat.INTERLEAVED)
            par = (row_v[pl.ds(off, 16)] & 1) == 1
            s16 = jnp.where(par, hi16, lo16).astype(jnp.float32)
            e16 = jnp.exp(s16 + bias_v[pl.ds(off, 16)])
            e_v[pl.ds(off, 16)] = e16
            plsc.addupdate_scatter(den_v, [col_v[pl.ds(off, 16)]], e16)

    _fire_a(0, sem_a)
    _fire_a(1, sem_b)

    def chunkA(g, carry):
        c = g * 2
        pltpu.make_async_copy(s_hbm.at[pl.ds(0, 128)],
                              s_v.at[pl.ds(0, 128)], sem_a).wait()
        _proc_a(c * 128)
        @pl.when(c + 2 < NCA)
        def _():
            _fire_a(c + 2, sem_a)
        pltpu.make_async_copy(s_hbm.at[pl.ds(0, 128)],
                              s_v.at[pl.ds(0, 128)], sem_b).wait()
        _proc_a((c + 1) * 128)
        @pl.when(c + 3 < NCA)
        def _():
            _fire_a(c + 3, sem_b)
        return carry
    lax.fori_loop(0, NCA // 2, chunkA, 0)

    NCB = EPT // 16

    def _fire_b(ci, buf, sm):
        pltpu.async_copy(v_hbm.at[row_v.at[pl.ds(ci * 16, 16)]], buf, sm)

    def _proc_b(base, buf):
        for j in range(16):
            ej = plsc.load_gather(
                e_v, [jnp.zeros((16,), jnp.int32) + (base + j)])
            for hh in range(2):
                for k in range(8):
                    buf[j, hh, pl.ds(k * 16, 16)] = (
                        buf[j, hh, pl.ds(k * 16, 16)] * ej)
        col64_v[...] = col_v[pl.ds(base, 16)]
        pltpu.sync_copy(buf, agg_sp.at[col64_v], add=True)

    _fire_b(0, vb_a, sem_a)
    _fire_b(1, vb_b, sem_b)

    def chunkB(g, carry):
        c = g * 2
        pltpu.make_async_copy(v_hbm.at[pl.ds(0, 16)], vb_a, sem_a).wait()
        _proc_b(c * 16, vb_a)
        @pl.when(c + 2 < NCB)
        def _():
            _fire_b(c + 2, vb_a, sem_a)
        pltpu.make_async_copy(v_hbm.at[pl.ds(0, 16)], vb_b, sem_b).wait()
        _proc_b((c + 1) * 16, vb_b)
        @pl.when(c + 3 < NCB)
        def _():
            _fire_b(c + 3, vb_b, sem_b)
        return carry
    lax.fori_loop(0, NCB // 2, chunkB, 0)

    pltpu.sync_copy(den_v, den_sp.at[s])
    plsc.subcore_barrier()
    pltpu.sync_copy(agg_sp.at[pl.ds(s * 256, 256)],
                    agg_out.at[c, pl.ds(s * 256, 256)])
    def dz(i, carry):
        acc_v[pl.ds(i * 16, 16)] = jnp.zeros((16,), jnp.float32)
        return carry
    lax.fori_loop(0, 16, dz, 0)
    for t in range(16):
        pltpu.sync_copy(den_sp.at[t, pl.ds(s * 256, 256)], tmp_v)
        for i in range(16):
            acc_v[pl.ds(i * 16, 16)] = (acc_v[pl.ds(i * 16, 16)] +
                                        tmp_v[pl.ds(i * 16, 16)])
    pltpu.sync_copy(acc_v, den_out.at[c, pl.ds(s * 256, 256)])


def kernel(x, edge_index, p_matrix, d_matrix, batch, params):
    row, col = edge_index[0], edge_index[1]
    scale = H ** (-0.5)
    row2d = row.reshape(NW, EPT)
    col2d = col.reshape(NW, EPT)
    fptab = jnp.concatenate(
        [jnp.pad(lp['fp'][:, 0], (0, 16 - lp['fp'].shape[0]))
         for lp in params['layers']])
    fdtab = jnp.concatenate([lp['fd'][:, 0] for lp in params['layers']])
    sidx, b0, b1, b2, b3 = _sc_prep(
        p_matrix.reshape(-1), d_matrix.reshape(-1), row2d, col2d,
        fptab, fdtab)
    biases = (b0, b1, b2, b3)

    h = _input_proj(x, params['W_in'], params['b_in'])
    for li, lp in enumerate(params['layers']):
        q, k, v = _qkv(h, lp['ln1_g'], lp['ln1_b'],
                       lp['wq'] * scale, lp['wk'], lp['wv'])
        S = _scores(q, k)
        s_pairs = jax.lax.bitcast_convert_type(
            S.reshape(N, N // 2, 2), jnp.int32).reshape(-1)
        agg_parts, den_parts = _sc_edge(
            s_pairs, v.reshape(N, 2, 128), row2d, col2d,
            sidx, biases[li])
        agg_parts = agg_parts.reshape(2, N, H)
        h = _ffn(h, agg_parts, den_parts, lp['ln2_g'], lp['ln2_b'],
                 lp['w1'], lp['b1'], lp['w2'], lp['b2'])
    return _head(h, batch, params['Wc1'], params['bc1'],
                 params['Wc2'], params['bc2'])


# speedup vs baseline: 1.7744x; 1.7744x over previous
import functools
import math

import jax
import jax.numpy as jnp
from jax import lax
from jax.experimental import pallas as pl
from jax.experimental.pallas import tpu as pltpu
from jax.experimental.pallas import tpu_sc as plsc

N = 4096
E = 131072
H = 256
G = 16
L = 4
ROW_BLK = 512
N_BLKS = N // ROW_BLK
NW = 32
EPT = E // NW
_SC_MESH = plsc.VectorSubcoreMesh(core_axis_name="c", subcore_axis_name="s")
_SC_PARAMS = pltpu.CompilerParams(needs_layout_passes=False)


def _ln_in(x, g, b):
    m = jnp.mean(x, axis=-1, keepdims=True)
    v = jnp.mean((x - m) ** 2, axis=-1, keepdims=True)
    return (x - m) * jax.lax.rsqrt(v + 1e-5) * g + b


def _dot3(a, b, dims):
    ah = a.astype(jnp.bfloat16)
    al = (a - ah.astype(jnp.float32)).astype(jnp.bfloat16)
    bh = b.astype(jnp.bfloat16)
    bl = (b - bh.astype(jnp.float32)).astype(jnp.bfloat16)

    def d(x, y):
        return jax.lax.dot_general(
            x, y, dims, precision=jax.lax.Precision.DEFAULT,
            preferred_element_type=jnp.float32)

    return d(ah, bh) + (d(ah, bl) + d(al, bh))


def _mm(a, b):
    return _dot3(a, b, (((1,), (0,)), ((), ())))


def _proj_body(x_ref, w_ref, b_ref, o_ref):
    o_ref[...] = _mm(x_ref[...], w_ref[...]) + b_ref[...]


def _input_proj(x, w, b):
    return pl.pallas_call(
        _proj_body,
        grid=(N_BLKS,),
        in_specs=[
            pl.BlockSpec((ROW_BLK, H), lambda i: (i, 0)),
            pl.BlockSpec((H, H), lambda i: (0, 0)),
            pl.BlockSpec((1, H), lambda i: (0, 0)),
        ],
        out_specs=pl.BlockSpec((ROW_BLK, H), lambda i: (i, 0)),
        out_shape=jax.ShapeDtypeStruct((N, H), jnp.float32),
    )(x, w, b.reshape(1, H))


def _qkv_body(h_ref, g_ref, b_ref, wq_ref, wk_ref, wv_ref,
              q_ref, k_ref, v_ref):
    xn = _ln_in(h_ref[...], g_ref[...], b_ref[...])
    q_ref[...] = _mm(xn, wq_ref[...])
    k_ref[...] = _mm(xn, wk_ref[...])
    v_ref[...] = _mm(xn, wv_ref[...])


def _qkv(h, g, b, wq, wk, wv):
    blk = pl.BlockSpec((ROW_BLK, H), lambda i: (i, 0))
    wblk = pl.BlockSpec((H, H), lambda i: (0, 0))
    vec = pl.BlockSpec((1, H), lambda i: (0, 0))
    return pl.pallas_call(
        _qkv_body,
        grid=(N_BLKS,),
        in_specs=[blk, vec, vec, wblk, wblk, wblk],
        out_specs=[blk, blk, blk],
        out_shape=[jax.ShapeDtypeStruct((N, H), jnp.float32)] * 3,
    )(h, g.reshape(1, H), b.reshape(1, H), wq, wk, wv)


def _score_body(q_ref, k_ref, s_ref):
    s_ref[...] = _dot3(q_ref[...], k_ref[...], (((1,), (1,)), ((), ())))


def _scores(q, k):
    return pl.pallas_call(
        _score_body,
        grid=(N_BLKS,),
        in_specs=[
            pl.BlockSpec((ROW_BLK, H), lambda i: (i, 0)),
            pl.BlockSpec((N, H), lambda i: (0, 0)),
        ],
        out_specs=pl.BlockSpec((ROW_BLK, N), lambda i: (i, 0)),
        out_shape=jax.ShapeDtypeStruct((N, N), jnp.float32),
    )(q, k)


def _ffn_body(h_ref, a0_ref, a1_ref, d0_ref, d1_ref, g_ref, b_ref,
              w1_ref, b1_ref, w2_ref, b2_ref, o_ref):
    den = d0_ref[0] + d1_ref[0] + 1e-16
    t = h_ref[...] + (a0_ref[0] + a1_ref[0]) / den.reshape(ROW_BLK, 1)
    xn = _ln_in(t, g_ref[...], b_ref[...])
    u = _mm(xn, w1_ref[...]) + b1_ref[...]
    act = 0.5 * u * (1.0 + jax.lax.erf(u * (2.0 ** -0.5)))
    o_ref[...] = t + _mm(act, w2_ref[...]) + b2_ref[...]


def _ffn(h, agg_parts, den_parts, g, b, w1, b1, w2, b2):
    blk = pl.BlockSpec((ROW_BLK, H), lambda i: (i, 0))
    vec = pl.BlockSpec((1, H), lambda i: (0, 0))
    return pl.pallas_call(
        _ffn_body,
        grid=(N_BLKS,),
        in_specs=[
            blk,
            pl.BlockSpec((1, ROW_BLK, H), lambda i: (0, i, 0)),
            pl.BlockSpec((1, ROW_BLK, H), lambda i: (0, i, 0)),
            pl.BlockSpec((1, ROW_BLK), lambda i: (0, i)),
            pl.BlockSpec((1, ROW_BLK), lambda i: (0, i)),
            vec, vec,
            pl.BlockSpec((H, 2 * H), lambda i: (0, 0)),
            pl.BlockSpec((1, 2 * H), lambda i: (0, 0)),
            pl.BlockSpec((2 * H, H), lambda i: (0, 0)),
            vec,
        ],
        out_specs=blk,
        out_shape=jax.ShapeDtypeStruct((N, H), jnp.float32),
    )(h, agg_parts[0][None], agg_parts[1][None],
      den_parts[0][None], den_parts[1][None],
      g.reshape(1, H), b.reshape(1, H),
      w1, b1.reshape(1, 2 * H), w2, b2.reshape(1, H))


def _ffnqkv_body(h_ref, a0_ref, a1_ref, d0_ref, d1_ref, g_ref, b_ref,
                 w1_ref, b1_ref, w2_ref, b2_ref,
                 g1_ref, bb1_ref, wq_ref, wk_ref, wv_ref,
                 o_ref, q_ref, k_ref, v_ref):
    den = d0_ref[0] + d1_ref[0] + 1e-16
    t = h_ref[...] + (a0_ref[0] + a1_ref[0]) / den.reshape(ROW_BLK, 1)
    xn = _ln_in(t, g_ref[...], b_ref[...])
    u = _mm(xn, w1_ref[...]) + b1_ref[...]
    act = 0.5 * u * (1.0 + jax.lax.erf(u * (2.0 ** -0.5)))
    hn = t + _mm(act, w2_ref[...]) + b2_ref[...]
    o_ref[...] = hn
    xn1 = _ln_in(hn, g1_ref[...], bb1_ref[...])
    q_ref[...] = _mm(xn1, wq_ref[...])
    k_ref[...] = _mm(xn1, wk_ref[...])
    v_ref[...] = _mm(xn1, wv_ref[...])


def _ffn_qkv(h, agg_parts, den_parts, g, b, w1, b1, w2, b2,
             g1, bb1, wq, wk, wv):
    blk = pl.BlockSpec((ROW_BLK, H), lambda i: (i, 0))
    vec = pl.BlockSpec((1, H), lambda i: (0, 0))
    wblk = pl.BlockSpec((H, H), lambda i: (0, 0))
    return pl.pallas_call(
        _ffnqkv_body,
        grid=(N_BLKS,),
        in_specs=[
            blk,
            pl.BlockSpec((1, ROW_BLK, H), lambda i: (0, i, 0)),
            pl.BlockSpec((1, ROW_BLK, H), lambda i: (0, i, 0)),
            pl.BlockSpec((1, ROW_BLK), lambda i: (0, i)),
            pl.BlockSpec((1, ROW_BLK), lambda i: (0, i)),
            vec, vec,
            pl.BlockSpec((H, 2 * H), lambda i: (0, 0)),
            pl.BlockSpec((1, 2 * H), lambda i: (0, 0)),
            pl.BlockSpec((2 * H, H), lambda i: (0, 0)),
            vec,
            vec, vec, wblk, wblk, wblk,
        ],
        out_specs=[blk, blk, blk, blk],
        out_shape=[jax.ShapeDtypeStruct((N, H), jnp.float32)] * 4,
    )(h, agg_parts[0][None], agg_parts[1][None],
      den_parts[0][None], den_parts[1][None],
      g.reshape(1, H), b.reshape(1, H),
      w1, b1.reshape(1, 2 * H), w2, b2.reshape(1, H),
      g1.reshape(1, H), bb1.reshape(1, H), wq, wk, wv)


def _projqkv_body(x_ref, w_ref, b_ref, g1_ref, bb1_ref,
                  wq_ref, wk_ref, wv_ref, o_ref, q_ref, k_ref, v_ref):
    hn = _mm(x_ref[...], w_ref[...]) + b_ref[...]
    o_ref[...] = hn
    xn1 = _ln_in(hn, g1_ref[...], bb1_ref[...])
    q_ref[...] = _mm(xn1, wq_ref[...])
    k_ref[...] = _mm(xn1, wk_ref[...])
    v_ref[...] = _mm(xn1, wv_ref[...])


def _proj_qkv(x, w, b, g1, bb1, wq, wk, wv):
    blk = pl.BlockSpec((ROW_BLK, H), lambda i: (i, 0))
    vec = pl.BlockSpec((1, H), lambda i: (0, 0))
    wblk = pl.BlockSpec((H, H), lambda i: (0, 0))
    return pl.pallas_call(
        _projqkv_body,
        grid=(N_BLKS,),
        in_specs=[blk, wblk, vec, vec, vec, wblk, wblk, wblk],
        out_specs=[blk, blk, blk, blk],
        out_shape=[jax.ShapeDtypeStruct((N, H), jnp.float32)] * 4,
    )(x, w, b.reshape(1, H), g1.reshape(1, H), bb1.reshape(1, H),
      wq, wk, wv)


def _head_body(h_ref, batch_ref, wc1_ref, bc1_ref, wc2_ref, bc2_ref, o_ref):
    onehot = (batch_ref[...] == jax.lax.broadcasted_iota(
        jnp.int32, (N, G), 1)).astype(jnp.float32)
    sums = jax.lax.dot_general(
        onehot, h_ref[...], (((0,), (0,)), ((), ())),
        precision=jax.lax.Precision.HIGHEST,
        preferred_element_type=jnp.float32)
    cnt = jnp.sum(onehot, axis=0, keepdims=True)
    gmean = sums / jnp.clip(cnt, 1.0, None).reshape(G, 1)
    z = jnp.maximum(_mm(gmean, wc1_ref[...]) + bc1_ref[...], 0.0)
    o_ref[...] = _mm(z, wc2_ref[...]) + bc2_ref[...]


def _head(h, batch, wc1, bc1, wc2, bc2):
    OUT = wc2.shape[1]
    return pl.pallas_call(
        _head_body,
        in_specs=[
            pl.BlockSpec((N, H), lambda: (0, 0)),
            pl.BlockSpec((N, 1), lambda: (0, 0)),
            pl.BlockSpec(wc1.shape, lambda: (0, 0)),
            pl.BlockSpec((1, wc1.shape[1]), lambda: (0, 0)),
            pl.BlockSpec(wc2.shape, lambda: (0, 0)),
            pl.BlockSpec((1, OUT), lambda: (0, 0)),
        ],
        out_specs=pl.BlockSpec((G, OUT), lambda: (0, 0)),
        out_shape=jax.ShapeDtypeStruct((G, OUT), jnp.float32),
    )(h, batch.reshape(N, 1), wc1, bc1.reshape(1, -1), wc2,
      bc2.reshape(1, -1))


import functools


@functools.partial(
    pl.kernel,
    out_type=[jax.ShapeDtypeStruct((NW, EPT), jnp.int32)] +
             [jax.ShapeDtypeStruct((NW, EPT), jnp.float32)] * L,
    mesh=_SC_MESH,
    scratch_types=[
        pltpu.VMEM((EPT,), jnp.int32),
        pltpu.VMEM((EPT,), jnp.int32),
        pltpu.VMEM((128,), jnp.int32),
        pltpu.VMEM((128,), jnp.int32),
        pltpu.VMEM((128,), jnp.int32),
        pltpu.VMEM((EPT,), jnp.int32),
        pltpu.VMEM((L * 16,), jnp.float32),
        pltpu.VMEM((L * 64,), jnp.float32),
        pltpu.VMEM((EPT,), jnp.float32),
        pltpu.VMEM((EPT,), jnp.float32),
        pltpu.VMEM((EPT,), jnp.float32),
        pltpu.VMEM((EPT,), jnp.float32),
        pltpu.SemaphoreType.DMA,
    ],
    compiler_params=_SC_PARAMS,
)
def _sc_prep(p_hbm, d_hbm, row_hbm, col_hbm, fp_hbm, fd_hbm,
             sidx_out, b0_out, b1_out, b2_out, b3_out,
             row_v, col_v, idx_v, p_v, d_v, sidx_v, fp_v, fd_v,
             b0_v, b1_v, b2_v, b3_v, sem):
    c = lax.axis_index("c")
    s = lax.axis_index("s")
    wid = c * 16 + s
    b_refs = (b0_v, b1_v, b2_v, b3_v)
    b_outs = (b0_out, b1_out, b2_out, b3_out)
    pltpu.sync_copy(row_hbm.at[wid], row_v)
    pltpu.sync_copy(col_hbm.at[wid], col_v)
    pltpu.sync_copy(fp_hbm, fp_v)
    pltpu.sync_copy(fd_hbm, fd_v)

    def chunk(ci, carry):
        base = ci * 128
        for g in range(8):
            off = base + g * 16
            r16 = row_v[pl.ds(off, 16)]
            c16 = col_v[pl.ds(off, 16)]
            idx_v[pl.ds(g * 16, 16)] = r16 * N + c16
            sidx_v[pl.ds(off, 16)] = c16 * N + r16
        pltpu.async_copy(p_hbm.at[idx_v], p_v, sem).wait()
        pltpu.async_copy(d_hbm.at[idx_v], d_v, sem).wait()
        for g in range(8):
            off = base + g * 16
            p16 = p_v[pl.ds(g * 16, 16)]
            d16 = d_v[pl.ds(g * 16, 16)]
            for l in range(L):
                bv = (plsc.load_gather(fp_v, [p16 + (l * 16)]) +
                      plsc.load_gather(fd_v, [d16 + (l * 64)]))
                b_refs[l][pl.ds(off, 16)] = bv
        return carry

    lax.fori_loop(0, EPT // 128, chunk, 0)
    pltpu.sync_copy(sidx_v, sidx_out.at[wid])
    for l in range(L):
        pltpu.sync_copy(b_refs[l], b_outs[l].at[wid])


_VCH = 64


@functools.partial(
    pl.kernel,
    out_type=[
        jax.ShapeDtypeStruct((2, N, 2, 128), jnp.float32),
        jax.ShapeDtypeStruct((2, N), jnp.float32),
    ],
    mesh=_SC_MESH,
    scratch_types=[
        pltpu.VMEM((EPT,), jnp.int32),
        pltpu.VMEM((EPT,), jnp.int32),
        pltpu.VMEM((EPT,), jnp.int32),
        pltpu.VMEM((16,), jnp.int32),
        pltpu.VMEM((EPT,), jnp.float32),
        pltpu.VMEM((EPT,), jnp.float32),
        pltpu.VMEM((EPT,), jnp.float32),
        pltpu.VMEM((N,), jnp.float32),
        pltpu.VMEM((16, 2, 128), jnp.float32),
        pltpu.VMEM((16, 2, 128), jnp.float32),
        pltpu.VMEM((256,), jnp.float32),
        pltpu.VMEM((256,), jnp.float32),
        pltpu.VMEM_SHARED((N, 2, 128), jnp.float32),
        pltpu.VMEM_SHARED((16, N), jnp.float32),
        pltpu.SemaphoreType.DMA,
        pltpu.SemaphoreType.DMA,
    ],
    compiler_params=_SC_PARAMS,
)
def _sc_edge(s_hbm, v_hbm, row_hbm, col_hbm, sidx_hbm, bias_hbm,
             agg_out, den_out,
             sidx_v, row_v, col_v, col64_v, bias_v, e_v, s_v, den_v,
             vb_a, vb_b, tmp_v, acc_v, agg_sp, den_sp, sem_a, sem_b):
    c = lax.axis_index("c")
    s = lax.axis_index("s")
    wid = c * 16 + s
    pltpu.sync_copy(sidx_hbm.at[wid], sidx_v)
    pltpu.sync_copy(row_hbm.at[wid], row_v)
    pltpu.sync_copy(col_hbm.at[wid], col_v)
    pltpu.sync_copy(bias_hbm.at[wid], bias_v)

    def zden(i, carry):
        den_v[pl.ds(i * 16, 16)] = jnp.zeros((16,), jnp.float32)
        return carry
    lax.fori_loop(0, N // 16, zden, 0)

    for i in range(16):
        for hh in range(2):
            for k in range(8):
                vb_a[i, hh, pl.ds(k * 16, 16)] = jnp.zeros(
                    (16,), jnp.float32)
    for k in range(256 // 16):
        pltpu.sync_copy(vb_a, agg_sp.at[pl.ds(s * 256 + k * 16, 16)])
    plsc.subcore_barrier()

    NCA = EPT // 128

    def _fire_a(ci, sm):
        pltpu.async_copy(s_hbm.at[sidx_v.at[pl.ds(ci * 128, 128)]],
                         s_v.at[pl.ds(ci * 128, 128)], sm)

    def _proc_a(base):
        for g in range(8):
            off = base + g * 16
            e16 = jnp.exp(s_v[pl.ds(off, 16)] + bias_v[pl.ds(off, 16)])
            e_v[pl.ds(off, 16)] = e16
            plsc.addupdate_scatter(den_v, [col_v[pl.ds(off, 16)]], e16)

    _fire_a(0, sem_a)
    _fire_a(1, sem_b)

    def chunkA(g, carry):
        c = g * 2
        pltpu.make_async_copy(s_hbm.at[pl.ds(0, 128)],
                              s_v.at[pl.ds(0, 128)], sem_a).wait()
        _proc_a(c * 128)
        @pl.when(c + 2 < NCA)
        def _():
            _fire_a(c + 2, sem_a)
        pltpu.make_async_copy(s_hbm.at[pl.ds(0, 128)],
                              s_v.at[pl.ds(0, 128)], sem_b).wait()
        _proc_a((c + 1) * 128)
        @pl.when(c + 3 < NCA)
        def _():
            _fire_a(c + 3, sem_b)
        return carry
    lax.fori_loop(0, NCA // 2, chunkA, 0)

    NCB = EPT // 16

    def _fire_b(ci, buf, sm):
        pltpu.async_copy(v_hbm.at[row_v.at[pl.ds(ci * 16, 16)]], buf, sm)

    def _proc_b(base, buf):
        for j in range(16):
            ej = plsc.load_gather(
                e_v, [jnp.zeros((16,), jnp.int32) + (base + j)])
            for hh in range(2):
                for k in range(8):
                    buf[j, hh, pl.ds(k * 16, 16)] = (
                        buf[j, hh, pl.ds(k * 16, 16)] * ej)
        col64_v[...] = col_v[pl.ds(base, 16)]
        pltpu.sync_copy(buf, agg_sp.at[col64_v], add=True)

    _fire_b(0, vb_a, sem_a)
    _fire_b(1, vb_b, sem_b)

    def chunkB(g, carry):
        c = g * 2
        pltpu.make_async_copy(v_hbm.at[pl.ds(0, 16)], vb_a, sem_a).wait()
        _proc_b(c * 16, vb_a)
        @pl.when(c + 2 < NCB)
        def _():
            _fire_b(c + 2, vb_a, sem_a)
        pltpu.make_async_copy(v_hbm.at[pl.ds(0, 16)], vb_b, sem_b).wait()
        _proc_b((c + 1) * 16, vb_b)
        @pl.when(c + 3 < NCB)
        def _():
            _fire_b(c + 3, vb_b, sem_b)
        return carry
    lax.fori_loop(0, NCB // 2, chunkB, 0)

    pltpu.sync_copy(den_v, den_sp.at[s])
    plsc.subcore_barrier()
    pltpu.sync_copy(agg_sp.at[pl.ds(s * 256, 256)],
                    agg_out.at[c, pl.ds(s * 256, 256)])
    def dz(i, carry):
        acc_v[pl.ds(i * 16, 16)] = jnp.zeros((16,), jnp.float32)
        return carry
    lax.fori_loop(0, 16, dz, 0)
    for t in range(16):
        pltpu.sync_copy(den_sp.at[t, pl.ds(s * 256, 256)], tmp_v)
        for i in range(16):
            acc_v[pl.ds(i * 16, 16)] = (acc_v[pl.ds(i * 16, 16)] +
                                        tmp_v[pl.ds(i * 16, 16)])
    pltpu.sync_copy(acc_v, den_out.at[c, pl.ds(s * 256, 256)])


def kernel(x, edge_index, p_matrix, d_matrix, batch, params):
    row, col = edge_index[0], edge_index[1]
    scale = H ** (-0.5)
    row2d = row.reshape(NW, EPT)
    col2d = col.reshape(NW, EPT)
    fptab = jnp.concatenate(
        [jnp.pad(lp['fp'][:, 0], (0, 16 - lp['fp'].shape[0]))
         for lp in params['layers']])
    fdtab = jnp.concatenate([lp['fd'][:, 0] for lp in params['layers']])
    sidx, b0, b1, b2, b3 = _sc_prep(
        p_matrix.reshape(-1), d_matrix.reshape(-1), row2d, col2d,
        fptab, fdtab)
    biases = (b0, b1, b2, b3)

    layers = params['layers']
    h, q, k, v = _proj_qkv(x, params['W_in'], params['b_in'],
                           layers[0]['ln1_g'], layers[0]['ln1_b'],
                           layers[0]['wq'] * scale, layers[0]['wk'],
                           layers[0]['wv'])
    for li, lp in enumerate(layers):
        S = _scores(q, k)
        agg_parts, den_parts = _sc_edge(
            S.reshape(-1), v.reshape(N, 2, 128), row2d, col2d,
            sidx, biases[li])
        agg_parts = agg_parts.reshape(2, N, H)
        if li + 1 < len(layers):
            nxt = layers[li + 1]
            h, q, k, v = _ffn_qkv(
                h, agg_parts, den_parts, lp['ln2_g'], lp['ln2_b'],
                lp['w1'], lp['b1'], lp['w2'], lp['b2'],
                nxt['ln1_g'], nxt['ln1_b'],
                nxt['wq'] * scale, nxt['wk'], nxt['wv'])
        else:
            h = _ffn(h, agg_parts, den_parts, lp['ln2_g'], lp['ln2_b'],
                     lp['w1'], lp['b1'], lp['w2'], lp['b2'])
    return _head(h, batch, params['Wc1'], params['bc1'],
                 params['Wc2'], params['bc2'])
